# baseline (speedup 1.0000x reference)
import functools

import jax
import jax.numpy as jnp
from jax import lax
from jax.experimental import pallas as pl
from jax.experimental.pallas import tpu as pltpu
from jax.experimental.pallas import tpu_sc as plsc

_NC = 2
_NS = 16
_NW = _NC * _NS
_L = 16

_CC = 1024


_GATHER_DNUMS = lax.GatherDimensionNumbers(
    offset_dims=(), collapsed_slice_dims=(0,), start_index_map=(0,)
)


def _lane_perm(v, idx):
    return lax.gather(
        v, idx[:, None], _GATHER_DNUMS, (1,),
        mode=lax.GatherScatterMode.PROMISE_IN_BOUNDS,
    )


def _sc_scan_chunk(in_ref, out_ref, carries):
    iota = lax.iota(jnp.int32, _L)
    shifts = [(jnp.maximum(iota - k, 0), iota >= k) for k in (1, 2, 4, 8)]
    last = jnp.full((_L,), _L - 1, jnp.int32)
    zero = jnp.float32(0)

    def step(j, carry):
        new = []
        for r in range(_L):
            v = in_ref[r, pl.ds(j * _L, _L)]
            cs = v
            for idx, m in shifts:
                cs = cs + jnp.where(m, _lane_perm(cs, idx), zero)
            cs = cs + carry[r]
            out_ref[r, pl.ds(j * _L, _L)] = cs
            new.append(_lane_perm(cs, last))
        return tuple(new)

    return lax.fori_loop(0, _CC // _L, step, carries)


def _sc_body(x_hbm, o_hbm, in_bufs, out_bufs, in_sems, out_sems, *, x_row0):
    rows, cols = o_hbm.shape
    rows_per_w = rows // _NW
    nch = cols // _CC
    wid = lax.axis_index("s") * _NC + lax.axis_index("c")
    row0 = wid * rows_per_w

    def start_in(c, p, r0):
        pltpu.async_copy(
            x_hbm.at[pl.ds(x_row0 + r0, _L), pl.ds(c * _CC, _CC)], in_bufs[p], in_sems[p]
        )

    def wait_in(c, p, r0):
        pltpu.make_async_copy(
            x_hbm.at[pl.ds(x_row0 + r0, _L), pl.ds(c * _CC, _CC)], in_bufs[p], in_sems[p]
        ).wait()

    def start_out(c, p, r0):
        pltpu.async_copy(
            out_bufs[p], o_hbm.at[pl.ds(r0, _L), pl.ds(c * _CC, _CC)], out_sems[p]
        )

    def wait_out(c, p, r0):
        pltpu.make_async_copy(
            out_bufs[p], o_hbm.at[pl.ds(r0, _L), pl.ds(c * _CC, _CC)], out_sems[p]
        ).wait()

    def do_group(g, _):
        r0 = row0 + g * _L
        start_in(0, 0, r0)
        carries = (jnp.zeros((_L,), jnp.float32),) * _L
        for c in range(nch):
            p = c % 2
            if c + 1 < nch:
                start_in(c + 1, 1 - p, r0)
            wait_in(c, p, r0)
            if c >= 2:
                wait_out(c - 2, p, r0)
            carries = _sc_scan_chunk(in_bufs[p], out_bufs[p], carries)
            start_out(c, p, r0)
        wait_out(nch - 2, nch % 2, r0)
        wait_out(nch - 1, (nch - 1) % 2, r0)
        return _

    lax.fori_loop(0, rows_per_w // _L, do_group, 0)


def _sc_call(x, out_rows, x_row0):
    cols = x.shape[1]
    mesh = plsc.VectorSubcoreMesh(core_axis_name="c", subcore_axis_name="s")
    buf = pltpu.VMEM((_L, _CC), jnp.float32)
    return pl.kernel(
        functools.partial(_sc_body, x_row0=x_row0),
        out_type=jax.ShapeDtypeStruct((out_rows, cols), x.dtype),
        mesh=mesh,
        scratch_types=[
            [buf] * 2,
            [buf] * 2,
            [pltpu.SemaphoreType.DMA] * 2,
            [pltpu.SemaphoreType.DMA] * 2,
        ],
        compiler_params=pltpu.CompilerParams(
            use_tc_tiling_on_sc=False, needs_layout_passes=False
        ),
    )(x)


@jax.jit
def kernel(x):
    rows, _ = x.shape
    return _sc_call(x, rows, 0)


def _tc_cumsum_body(x_ref, o_ref, *, cb: int):
    rb, cols = x_ref.shape
    nchunk = cols // cb
    row = lax.broadcasted_iota(jnp.int32, (cb, cb), 0)
    col = lax.broadcasted_iota(jnp.int32, (cb, cb), 1)
    tri = (row <= col).astype(jnp.float32)

    carry = jnp.zeros((rb, 1), jnp.float32)
    for c in range(nchunk):
        blk = x_ref[:, c * cb : (c + 1) * cb]
        cs = lax.dot(blk, tri, preferred_element_type=jnp.float32)
        o_ref[:, c * cb : (c + 1) * cb] = cs + carry
        carry = carry + cs[:, cb - 1 : cb]


def _tc_call(x, r_tc):
    rows, cols = x.shape
    rb = 256
    cb = 256
    body = functools.partial(_tc_cumsum_body, cb=cb)
    return pl.pallas_call(
        body,
        grid=(r_tc // rb,),
        in_specs=[pl.BlockSpec((rb, cols), lambda i: (i, 0))],
        out_specs=pl.BlockSpec((rb, cols), lambda i: (i, 0)),
        out_shape=jax.ShapeDtypeStruct((rows, cols), x.dtype),
    )(x)


# speedup vs baseline: 1.3109x; 1.3109x over previous
import functools

import jax
import jax.numpy as jnp
from jax import lax
from jax.experimental import pallas as pl
from jax.experimental.pallas import tpu as pltpu
from jax.experimental.pallas import tpu_sc as plsc

_NC = 2
_NS = 16
_NW = _NC * _NS
_L = 16

_CC = 1024


def _sc_scan_chunk(in_ref, out_ref, carries):
    def step(j, carry):
        new = []
        for r in range(_L):
            v = in_ref[r, pl.ds(j * _L, _L)]
            cs = plsc.cumsum(v) + carry[r]
            out_ref[r, pl.ds(j * _L, _L)] = cs
            new.append(cs[_L - 1])
        return tuple(new)

    return lax.fori_loop(0, _CC // _L, step, carries)


def _sc_body(x_hbm, o_hbm, in_bufs, out_bufs, in_sems, out_sems, *, row_base, n_rows):
    rows, cols = o_hbm.shape
    rows_per_w = n_rows // _NW
    nch = cols // _CC
    wid = lax.axis_index("s") * _NC + lax.axis_index("c")
    row0 = row_base + wid * rows_per_w

    def start_in(c, p, r0):
        pltpu.async_copy(
            x_hbm.at[pl.ds(r0, _L), pl.ds(c * _CC, _CC)], in_bufs[p], in_sems[p]
        )

    def wait_in(c, p, r0):
        pltpu.make_async_copy(
            x_hbm.at[pl.ds(r0, _L), pl.ds(c * _CC, _CC)], in_bufs[p], in_sems[p]
        ).wait()

    def start_out(c, p, r0):
        pltpu.async_copy(
            out_bufs[p], o_hbm.at[pl.ds(r0, _L), pl.ds(c * _CC, _CC)], out_sems[p]
        )

    def wait_out(c, p, r0):
        pltpu.make_async_copy(
            out_bufs[p], o_hbm.at[pl.ds(r0, _L), pl.ds(c * _CC, _CC)], out_sems[p]
        ).wait()

    def do_group(g, _):
        r0 = row0 + g * _L
        start_in(0, 0, r0)
        carries = (jnp.float32(0.0),) * _L
        for c in range(nch):
            p = c % 2
            if c + 1 < nch:
                start_in(c + 1, 1 - p, r0)
            wait_in(c, p, r0)
            if c >= 2:
                wait_out(c - 2, p, r0)
            carries = _sc_scan_chunk(in_bufs[p], out_bufs[p], carries)
            start_out(c, p, r0)
        wait_out(nch - 2, nch % 2, r0)
        wait_out(nch - 1, (nch - 1) % 2, r0)
        return _

    lax.fori_loop(0, rows_per_w // _L, do_group, 0)


def _sc_call(x, row_base, n_rows):
    rows, cols = x.shape
    mesh = plsc.VectorSubcoreMesh(core_axis_name="c", subcore_axis_name="s")
    buf = pltpu.VMEM((_L, _CC), jnp.float32)
    return pl.kernel(
        functools.partial(_sc_body, row_base=row_base, n_rows=n_rows),
        out_type=jax.ShapeDtypeStruct((rows, cols), x.dtype),
        mesh=mesh,
        scratch_types=[
            [buf] * 2,
            [buf] * 2,
            [pltpu.SemaphoreType.DMA] * 2,
            [pltpu.SemaphoreType.DMA] * 2,
        ],
        compiler_params=pltpu.CompilerParams(
            use_tc_tiling_on_sc=False, needs_layout_passes=False
        ),
    )(x)


_R_SC = 512


@jax.jit
def kernel(x):
    rows, _ = x.shape
    r_tc = rows - _R_SC
    sc_out = _sc_call(x, r_tc, _R_SC)
    return _tc_call(x, sc_out, r_tc)


def _tc_cumsum_body(x_ref, prev_ref, o_ref, *, cb: int):
    del prev_ref
    rb, cols = x_ref.shape
    nchunk = cols // cb
    row = lax.broadcasted_iota(jnp.int32, (cb, cb), 0)
    col = lax.broadcasted_iota(jnp.int32, (cb, cb), 1)
    tri = (row <= col).astype(jnp.float32)

    carry = jnp.zeros((rb, 1), jnp.float32)
    for c in range(nchunk):
        blk = x_ref[:, c * cb : (c + 1) * cb]
        cs = lax.dot(blk, tri, preferred_element_type=jnp.float32)
        o_ref[:, c * cb : (c + 1) * cb] = cs + carry
        carry = carry + cs[:, cb - 1 : cb]


def _tc_call(x, prev, r_tc):
    rows, cols = x.shape
    rb = 256
    cb = 256
    body = functools.partial(_tc_cumsum_body, cb=cb)
    return pl.pallas_call(
        body,
        grid=(r_tc // rb,),
        in_specs=[
            pl.BlockSpec((rb, cols), lambda i: (i, 0)),
            pl.BlockSpec(memory_space=pl.ANY),
        ],
        out_specs=pl.BlockSpec((rb, cols), lambda i: (i, 0)),
        out_shape=jax.ShapeDtypeStruct((rows, cols), x.dtype),
        input_output_aliases={1: 0},
    )(x, prev)


# speedup vs baseline: 1.9904x; 1.5183x over previous
import functools

import jax
import jax.numpy as jnp
from jax import lax
from jax.experimental import pallas as pl
from jax.experimental.pallas import tpu as pltpu
from jax.experimental.pallas import tpu_sc as plsc

_NC = 2
_NS = 16
_NW = _NC * _NS
_L = 16

_CC = 1024


def _sc_scan_chunk(in_ref, out_ref, carries):
    def step(j, carry):
        new = []
        for r in range(_L):
            v = in_ref[r, pl.ds(j * _L, _L)]
            cs = plsc.cumsum(v) + carry[r]
            out_ref[r, pl.ds(j * _L, _L)] = cs
            new.append(cs[_L - 1])
        return tuple(new)

    return lax.fori_loop(0, _CC // _L, step, carries)


def _sc_body(x_hbm, o_hbm, in_bufs, out_bufs, in_sems, out_sems, *, row_base):
    n_rows, cols = o_hbm.shape
    rows_per_w = n_rows // _NW
    nch = cols // _CC
    wid = lax.axis_index("s") * _NC + lax.axis_index("c")
    row0 = wid * rows_per_w

    def start_in(c, p, r0):
        pltpu.async_copy(
            x_hbm.at[pl.ds(row_base + r0, _L), pl.ds(c * _CC, _CC)], in_bufs[p], in_sems[p]
        )

    def wait_in(c, p, r0):
        pltpu.make_async_copy(
            x_hbm.at[pl.ds(row_base + r0, _L), pl.ds(c * _CC, _CC)], in_bufs[p], in_sems[p]
        ).wait()

    def start_out(c, p, r0):
        pltpu.async_copy(
            out_bufs[p], o_hbm.at[pl.ds(r0, _L), pl.ds(c * _CC, _CC)], out_sems[p]
        )

    def wait_out(c, p, r0):
        pltpu.make_async_copy(
            out_bufs[p], o_hbm.at[pl.ds(r0, _L), pl.ds(c * _CC, _CC)], out_sems[p]
        ).wait()

    def do_group(g, _):
        r0 = row0 + g * _L
        start_in(0, 0, r0)
        carries = (jnp.float32(0.0),) * _L
        for c in range(nch):
            p = c % 2
            if c + 1 < nch:
                start_in(c + 1, 1 - p, r0)
            wait_in(c, p, r0)
            if c >= 2:
                wait_out(c - 2, p, r0)
            carries = _sc_scan_chunk(in_bufs[p], out_bufs[p], carries)
            start_out(c, p, r0)
        wait_out(nch - 2, nch % 2, r0)
        wait_out(nch - 1, (nch - 1) % 2, r0)
        return _

    lax.fori_loop(0, rows_per_w // _L, do_group, 0)


def _sc_call(x, row_base, n_rows):
    _, cols = x.shape
    mesh = plsc.VectorSubcoreMesh(core_axis_name="c", subcore_axis_name="s")
    buf = pltpu.VMEM((_L, _CC), jnp.float32)
    return pl.kernel(
        functools.partial(_sc_body, row_base=row_base),
        out_type=jax.ShapeDtypeStruct((n_rows, cols), x.dtype),
        mesh=mesh,
        scratch_types=[
            [buf] * 2,
            [buf] * 2,
            [pltpu.SemaphoreType.DMA] * 2,
            [pltpu.SemaphoreType.DMA] * 2,
        ],
        compiler_params=pltpu.CompilerParams(
            use_tc_tiling_on_sc=False, needs_layout_passes=False
        ),
    )(x)


_R_SC = 512


@jax.jit
def kernel(x):
    rows, _ = x.shape
    r_tc = rows - _R_SC
    sc_out = _sc_call(x, r_tc, _R_SC)
    tc_out = _tc_call(x, r_tc)
    return lax.dynamic_update_slice(tc_out, sc_out, (r_tc, 0))


def _tc_cumsum_body(x_ref, o_ref, *, cb: int):
    rb, cols = x_ref.shape
    nchunk = cols // cb
    row = lax.broadcasted_iota(jnp.int32, (cb, cb), 0)
    col = lax.broadcasted_iota(jnp.int32, (cb, cb), 1)
    tri = (row <= col).astype(jnp.float32)

    carry = jnp.zeros((rb, 1), jnp.float32)
    for c in range(nchunk):
        blk = x_ref[:, c * cb : (c + 1) * cb]
        cs = lax.dot(blk, tri, preferred_element_type=jnp.float32)
        o_ref[:, c * cb : (c + 1) * cb] = cs + carry
        carry = carry + cs[:, cb - 1 : cb]


def _tc_call(x, r_tc):
    rows, cols = x.shape
    rb = 256
    cb = 256
    body = functools.partial(_tc_cumsum_body, cb=cb)
    return pl.pallas_call(
        body,
        grid=(r_tc // rb,),
        in_specs=[pl.BlockSpec((rb, cols), lambda i: (i, 0))],
        out_specs=pl.BlockSpec((rb, cols), lambda i: (i, 0)),
        out_shape=jax.ShapeDtypeStruct((rows, cols), x.dtype),
    )(x)
